# Initial kernel scaffold; baseline (speedup 1.0000x reference)
import jax
import jax.numpy as jnp
from jax.experimental import pallas as pl


def kernel(x, edge_index, W1, a_src1, a_dst1, b1, W2, a_src2, a_dst2, b2):
    raise NotImplementedError("write your pallas kernel here")


# speedup vs baseline: 41.5155x; 41.5155x over previous
import jax
import jax.numpy as jnp
from jax import lax
from jax.experimental import pallas as pl
from jax.experimental.pallas import tpu as pltpu
from jax.experimental.pallas import tpu_sc as plsc

N = 10000
D = 128
H1, C1 = 8, 8
C2 = 7

NC, NS, L = 2, 16, 16
NW = NC * NS
CHUNK = 128
EPT = 10240
EPAD = NW * EPT
NCHUNK = EPT // CHUNK
ROWS1 = 80
ROWS2 = 16
NPAD = 10240
TPAD = 10016
RPS = NPAD // NS
RCH = 128


def _lrelu(x):
    return jnp.where(x > 0, x, 0.2 * x)


def _make_edge_pass(rows, layer):

    def body(src_tab, dst_tab, src_idx, dst_idx, out, idx_s, idx_d, rows_s,
             rows_d, msg, sem_s, sem_d, acc):
        cid = lax.axis_index("c")
        sid = lax.axis_index("s")
        wid = sid * NC + cid

        def acc_body():
            groups = rows // L

            def zloop(t, _):
                zero16 = (lax.iota(jnp.int32, L) * 0).astype(jnp.float32)
                msg[t // groups, pl.ds((t % groups) * L, L)] = zero16
                return 0

            lax.fori_loop(0, RCH * groups, zloop, 0)

            def zcopy(t, _):
                pltpu.sync_copy(msg, acc.at[pl.ds(sid * RPS + t * RCH, RCH)])
                return 0

            lax.fori_loop(0, RPS // RCH, zcopy, 0)
            plsc.subcore_barrier()

            def chunk_body(c, _):
                base = wid * EPT + c * CHUNK
                pltpu.sync_copy(src_idx.at[pl.ds(base, CHUNK)], idx_s)
                pltpu.sync_copy(dst_idx.at[pl.ds(base, CHUNK)], idx_d)
                pltpu.async_copy(src_tab.at[idx_s], rows_s, sem_s)
                pltpu.async_copy(dst_tab.at[idx_d], rows_d, sem_d).wait()
                pltpu.make_async_copy(src_tab.at[idx_s], rows_s, sem_s).wait()

                def edge_body1(e, _):
                    lane = lax.iota(jnp.int32, L)
                    a16 = rows_s[e, pl.ds(rows - 16, L)]
                    b16 = rows_d[e, pl.ds(0, L)]
                    w16 = jnp.exp(_lrelu(a16 + b16))
                    msg[e, pl.ds(rows - 16, L)] = w16
                    erow = lane * 0 + e
                    half = (lane >= 8).astype(jnp.int32)
                    for r in range(4):
                        bidx = 2 * r + half + (rows - 16)
                        wr = plsc.load_gather(msg, [erow, bidx])
                        msg[e, pl.ds(r * L, L)] = rows_s[e, pl.ds(r * L, L)] * wr
                    return 0

                def edge_body2(e, _):
                    lane = lax.iota(jnp.int32, L)
                    a16 = rows_s[e, pl.ds(0, L)]
                    b16 = rows_d[e, pl.ds(0, L)]
                    w16 = jnp.exp(_lrelu(a16 + b16))
                    rows_d[e, pl.ds(0, L)] = w16
                    erow = lane * 0 + e
                    wb = plsc.load_gather(rows_d, [erow, lane * 0 + 8])
                    msg[e, pl.ds(0, L)] = wb * a16
                    return 0

                lax.fori_loop(0, CHUNK,
                              edge_body1 if layer == 1 else edge_body2, 0)
                pltpu.sync_copy(msg, acc.at[idx_d], add=True)
                return 0

            lax.fori_loop(0, NCHUNK, chunk_body, 0)
            plsc.subcore_barrier()

            def drain(t, _):
                r0 = sid * RPS + t * RCH
                pltpu.sync_copy(acc.at[pl.ds(r0, RCH)], msg)
                pltpu.sync_copy(msg, out.at[cid, pl.ds(r0, RCH)])
                return 0

            lax.fori_loop(0, RPS // RCH, drain, 0)

        acc_body()

    mesh = plsc.VectorSubcoreMesh(core_axis_name="c", subcore_axis_name="s")
    return pl.kernel(
        body,
        out_type=jax.ShapeDtypeStruct((NC, NPAD, rows), jnp.float32),
        mesh=mesh,
        scratch_types=[
            pltpu.VMEM((CHUNK,), jnp.int32),
            pltpu.VMEM((CHUNK,), jnp.int32),
            pltpu.VMEM((CHUNK, rows), jnp.float32),
            pltpu.VMEM((CHUNK, 16), jnp.float32),
            pltpu.VMEM((CHUNK, rows), jnp.float32),
            pltpu.SemaphoreType.DMA,
            pltpu.SemaphoreType.DMA,
            pltpu.VMEM_SHARED((NPAD, rows), jnp.float32),
        ],
        compiler_params=pltpu.CompilerParams(use_tc_tiling_on_sc=False,
                                             needs_layout_passes=False),
    )


_edge_pass1 = _make_edge_pass(ROWS1, 1)
_edge_pass2 = _make_edge_pass(ROWS2, 2)


def _rep8(blk):
    del blk
    hh = lax.broadcasted_iota(jnp.int32, (8, 64), 0)
    cc = lax.broadcasted_iota(jnp.int32, (8, 64), 1) // 8
    return (hh == cc).astype(jnp.float32)


def _stage_a_body(x_ref, w_ref, out_ref):
    out_ref[...] = jnp.dot(x_ref[...], w_ref[...],
                           preferred_element_type=jnp.float32)


def _stage_a(x, wcat):
    return pl.pallas_call(
        _stage_a_body,
        out_shape=jax.ShapeDtypeStruct((N, 96), jnp.float32),
    )(x, wcat)


def _stage_c_body(acca_ref, accb_ref, hext_ref, adext_ref, w2f_ref, b1_ref,
                  out_ref):
    acc = acca_ref[...] + accb_ref[...]
    h = hext_ref[:, :64]
    a_s = hext_ref[:, 64:72]
    a_d = adext_ref[:, :8]
    wself = jnp.exp(_lrelu(a_s + a_d))
    rep = _rep8(None)
    wrep = jnp.dot(wself, rep, preferred_element_type=jnp.float32)
    den = acc[:, 64:72] + wself
    drep = jnp.dot(den, rep, preferred_element_type=jnp.float32)
    num = acc[:, :64] + h * wrep
    x2 = num / (drep + 1e-16) + b1_ref[...]
    x2 = jnp.where(x2 > 0, x2, jnp.exp(jnp.minimum(x2, 0.0)) - 1.0)
    o = jnp.dot(x2, w2f_ref[...], preferred_element_type=jnp.float32)
    col = lax.broadcasted_iota(jnp.int32, o.shape, 1)
    out_ref[...] = o + (col == 7).astype(jnp.float32)


def _stage_c(acca, accb, hext, adext, w2f, b1):
    return pl.pallas_call(
        _stage_c_body,
        out_shape=jax.ShapeDtypeStruct((N, 32), jnp.float32),
        grid=(5,),
        in_specs=[
            pl.BlockSpec((N // 5, 80), lambda i: (i, 0)),
            pl.BlockSpec((N // 5, 80), lambda i: (i, 0)),
            pl.BlockSpec((N // 5, 80), lambda i: (i, 0)),
            pl.BlockSpec((N // 5, 16), lambda i: (i, 0)),
            pl.BlockSpec((64, 32), lambda i: (0, 0)),
            pl.BlockSpec((1, 64), lambda i: (0, 0)),
        ],
        out_specs=pl.BlockSpec((N // 5, 32), lambda i: (i, 0)),
    )(acca, accb, hext, adext, w2f, b1)


def _stage_e_body(acca_ref, accb_ref, h2ext_ref, ad2ext_ref, b2_ref, out_ref):
    acc = acca_ref[...] + accb_ref[...]
    h2 = h2ext_ref[:, :7]
    wself = jnp.exp(_lrelu(h2ext_ref[:, 8:9] + ad2ext_ref[:, 0:1]))
    num = acc[:, :7] + wself * h2
    den = acc[:, 7:8] + wself
    o = num / (den + 1e-16) + b2_ref[...]
    o = o - jnp.max(o, axis=1, keepdims=True)
    out_ref[...] = o - jnp.log(jnp.sum(jnp.exp(o), axis=1, keepdims=True))


def _stage_e(acca, accb, h2ext, ad2ext, b2):
    return pl.pallas_call(
        _stage_e_body,
        out_shape=jax.ShapeDtypeStruct((N, 7), jnp.float32),
        grid=(5,),
        in_specs=[
            pl.BlockSpec((N // 5, 16), lambda i: (i, 0)),
            pl.BlockSpec((N // 5, 16), lambda i: (i, 0)),
            pl.BlockSpec((N // 5, 16), lambda i: (i, 0)),
            pl.BlockSpec((N // 5, 16), lambda i: (i, 0)),
            pl.BlockSpec((1, 7), lambda i: (0, 0)),
        ],
        out_specs=pl.BlockSpec((N // 5, 7), lambda i: (i, 0)),
    )(acca, accb, h2ext, ad2ext, b2)


@jax.jit
def kernel(x, edge_index, W1, a_src1, a_dst1, b1, W2, a_src2, a_dst2, b2):
    f32 = jnp.float32
    eye8 = jnp.eye(H1, dtype=f32)
    As = (eye8[:, None, :] * a_src1[:, :, None]).reshape(H1 * C1, H1)
    Ad = (eye8[:, None, :] * a_dst1[:, :, None]).reshape(H1 * C1, H1)
    wcat = jnp.concatenate(
        [W1, W1 @ As, W1 @ As, W1 @ Ad, W1 @ Ad], axis=1)
    v_s2 = W2 @ a_src2[0]
    v_d2 = W2 @ a_dst2[0]
    w2f = jnp.concatenate(
        [W2, jnp.zeros((H1 * C1, 1), f32),
         jnp.tile(v_s2[:, None], (1, 8)),
         jnp.tile(v_d2[:, None], (1, 16))], axis=1)

    e_in = edge_index.shape[1]
    src = edge_index[0].astype(jnp.int32)
    dst = edge_index[1].astype(jnp.int32)
    src_p = jnp.concatenate([src, jnp.zeros((EPAD - e_in,), jnp.int32)])
    dst_p = jnp.concatenate([dst, jnp.full((EPAD - e_in,), N, jnp.int32)])

    packed1 = _stage_a(x, wcat)
    hext1 = jnp.pad(packed1[:, :80], ((0, TPAD - N), (0, 0)))
    adext1 = jnp.pad(packed1[:, 80:96], ((0, TPAD - N), (0, 0)))
    acc1 = _edge_pass1(hext1, adext1, src_p, dst_p)
    packed2 = _stage_c(acc1[0, :N], acc1[1, :N], hext1[:N], adext1[:N], w2f,
                       b1.reshape(1, 64))
    h2ext = jnp.pad(packed2[:, :16], ((0, TPAD - N), (0, 0)))
    ad2ext = jnp.pad(packed2[:, 16:32], ((0, TPAD - N), (0, 0)))

    acc2 = _edge_pass2(h2ext, ad2ext, src_p, dst_p)
    return _stage_e(acc2[0, :N], acc2[1, :N], h2ext[:N], ad2ext[:N],
                    b2.reshape(1, 7))


# speedup vs baseline: 105.3649x; 2.5380x over previous
import jax
import jax.numpy as jnp
from jax import lax
from jax.experimental import pallas as pl
from jax.experimental.pallas import tpu as pltpu
from jax.experimental.pallas import tpu_sc as plsc

N = 10000
D = 128
H1, C1 = 8, 8
C2 = 7

NC, NS, L = 2, 16, 16
NW = NC * NS
CHUNK = 128
EPT = 10240
EPAD = NW * EPT
NCHUNK = EPT // CHUNK
ROWS1 = 80
ROWS2 = 16
NPAD = 10240
TPAD = 10016
RPS = NPAD // NS
RCH = 128


def _lrelu(x):
    return jnp.where(x > 0, x, 0.2 * x)


def _make_edge_pass(rows, layer):

    def body(src_tab, dst_tab, src_idx, dst_idx, out, idx_s, idx_d, rows_s0,
             rows_s1, rows_d0, rows_d1, msg0, msg1, wbuf, sem_s0, sem_s1,
             sem_d0, sem_d1, acc):
        cid = lax.axis_index("c")
        sid = lax.axis_index("s")
        wid = sid * NC + cid
        rows_s = (rows_s0, rows_s1)
        rows_d = (rows_d0, rows_d1)
        msg = (msg0, msg1)
        sem_s = (sem_s0, sem_s1)
        sem_d = (sem_d0, sem_d1)

        def acc_body():
            groups = rows // L

            def zloop(t, _):
                zero16 = (lax.iota(jnp.int32, L) * 0).astype(jnp.float32)
                msg0[t // groups, pl.ds((t % groups) * L, L)] = zero16
                return 0

            lax.fori_loop(0, RCH * groups, zloop, 0)

            def zcopy(t, _):
                pltpu.sync_copy(msg0, acc.at[pl.ds(sid * RPS + t * RCH, RCH)])
                return 0

            lax.fori_loop(0, RPS // RCH, zcopy, 0)

            pltpu.sync_copy(src_idx.at[wid], idx_s)
            pltpu.sync_copy(dst_idx.at[wid], idx_d)
            plsc.subcore_barrier()

            def fire(c, par):
                pltpu.async_copy(src_tab.at[idx_s.at[c]], rows_s[par],
                                 sem_s[par])
                pltpu.async_copy(dst_tab.at[idx_d.at[c]], rows_d[par],
                                 sem_d[par])

            def wait(c, par):
                pltpu.make_async_copy(src_tab.at[idx_s.at[c]], rows_s[par],
                                      sem_s[par]).wait()
                pltpu.make_async_copy(dst_tab.at[idx_d.at[c]], rows_d[par],
                                      sem_d[par]).wait()

            def compute(c, par):
                rs, rd, mg = rows_s[par], rows_d[par], msg[par]

                if layer == 1:
                    def wpass(e, _):
                        a16 = rd[e, pl.ds(0, L)]
                        b16 = rs[e, pl.ds(rows - 16, L)]
                        w16 = jnp.exp(_lrelu(a16 + b16))
                        wbuf[e, pl.ds(0, L)] = w16
                        mg[e, pl.ds(rows - 16, L)] = w16
                        return 0

                    def mpass(e, _):
                        lane = lax.iota(jnp.int32, L)
                        erow = lane * 0 + e
                        half = (lane >= 8).astype(jnp.int32)
                        for r in range(4):
                            wr = plsc.load_gather(wbuf, [erow, 2 * r + half])
                            mg[e, pl.ds(r * L, L)] = rs[e, pl.ds(r * L, L)] * wr
                        return 0
                else:
                    def wpass(e, _):
                        a16 = rs[e, pl.ds(0, L)]
                        b16 = rd[e, pl.ds(0, L)]
                        w16 = jnp.exp(_lrelu(a16 + b16))
                        wbuf[e, pl.ds(0, L)] = w16
                        return 0

                    def mpass(e, _):
                        lane = lax.iota(jnp.int32, L)
                        erow = lane * 0 + e
                        wb = plsc.load_gather(wbuf, [erow, lane * 0 + 8])
                        mg[e, pl.ds(0, L)] = wb * rs[e, pl.ds(0, L)]
                        return 0

                plsc.parallel_loop(0, CHUNK, unroll=4)(
                    lambda e: (wpass(e, 0), None)[1])
                plsc.parallel_loop(0, CHUNK, unroll=4)(
                    lambda e: (mpass(e, 0), None)[1])
                pltpu.sync_copy(mg, acc.at[idx_d.at[c]], add=True)

            def phase(c, par):
                @pl.when(c + 1 < NCHUNK)
                def _():
                    fire(c + 1, 1 - par)

                wait(c, par)
                compute(c, par)

            fire(0, 0)

            def pair_body(q, _):
                phase(2 * q, 0)
                phase(2 * q + 1, 1)
                return 0

            lax.fori_loop(0, NCHUNK // 2, pair_body, 0)
            plsc.subcore_barrier()

            def drain(t, _):
                r0 = sid * RPS + t * RCH
                pltpu.sync_copy(acc.at[pl.ds(r0, RCH)], msg0)
                pltpu.sync_copy(msg0, out.at[cid, pl.ds(r0, RCH)])
                return 0

            lax.fori_loop(0, RPS // RCH, drain, 0)

        acc_body()

    mesh = plsc.VectorSubcoreMesh(core_axis_name="c", subcore_axis_name="s")
    return pl.kernel(
        body,
        out_type=jax.ShapeDtypeStruct((NC, NPAD, rows), jnp.float32),
        mesh=mesh,
        scratch_types=[
            pltpu.VMEM((NCHUNK, CHUNK), jnp.int32),
            pltpu.VMEM((NCHUNK, CHUNK), jnp.int32),
            pltpu.VMEM((CHUNK, rows), jnp.float32),
            pltpu.VMEM((CHUNK, rows), jnp.float32),
            pltpu.VMEM((CHUNK, 16), jnp.float32),
            pltpu.VMEM((CHUNK, 16), jnp.float32),
            pltpu.VMEM((CHUNK, rows), jnp.float32),
            pltpu.VMEM((CHUNK, rows), jnp.float32),
            pltpu.VMEM((CHUNK, 16), jnp.float32),
            pltpu.SemaphoreType.DMA,
            pltpu.SemaphoreType.DMA,
            pltpu.SemaphoreType.DMA,
            pltpu.SemaphoreType.DMA,
            pltpu.VMEM_SHARED((NPAD, rows), jnp.float32),
        ],
        compiler_params=pltpu.CompilerParams(use_tc_tiling_on_sc=False,
                                             needs_layout_passes=False),
    )


_edge_pass1 = _make_edge_pass(ROWS1, 1)
_edge_pass2 = _make_edge_pass(ROWS2, 2)


def _rep8(blk):
    del blk
    hh = lax.broadcasted_iota(jnp.int32, (8, 64), 0)
    cc = lax.broadcasted_iota(jnp.int32, (8, 64), 1) // 8
    return (hh == cc).astype(jnp.float32)


def _stage_a_body(x_ref, w_ref, out_ref):
    out_ref[...] = jnp.dot(x_ref[...], w_ref[...],
                           preferred_element_type=jnp.float32)


def _stage_a(x, wcat):
    return pl.pallas_call(
        _stage_a_body,
        out_shape=jax.ShapeDtypeStruct((N, 96), jnp.float32),
    )(x, wcat)


def _stage_c_body(acca_ref, accb_ref, hext_ref, adext_ref, w2f_ref, b1_ref,
                  out_ref):
    acc = acca_ref[...] + accb_ref[...]
    h = hext_ref[:, :64]
    a_s = hext_ref[:, 64:72]
    a_d = adext_ref[:, :8]
    wself = jnp.exp(_lrelu(a_s + a_d))
    rep = _rep8(None)
    wrep = jnp.dot(wself, rep, preferred_element_type=jnp.float32)
    den = acc[:, 64:72] + wself
    drep = jnp.dot(den, rep, preferred_element_type=jnp.float32)
    num = acc[:, :64] + h * wrep
    x2 = num / (drep + 1e-16) + b1_ref[...]
    x2 = jnp.where(x2 > 0, x2, jnp.exp(jnp.minimum(x2, 0.0)) - 1.0)
    o = jnp.dot(x2, w2f_ref[...], preferred_element_type=jnp.float32)
    col = lax.broadcasted_iota(jnp.int32, o.shape, 1)
    out_ref[...] = o + (col == 7).astype(jnp.float32)


def _stage_c(acca, accb, hext, adext, w2f, b1):
    return pl.pallas_call(
        _stage_c_body,
        out_shape=jax.ShapeDtypeStruct((N, 32), jnp.float32),
        grid=(5,),
        in_specs=[
            pl.BlockSpec((N // 5, 80), lambda i: (i, 0)),
            pl.BlockSpec((N // 5, 80), lambda i: (i, 0)),
            pl.BlockSpec((N // 5, 80), lambda i: (i, 0)),
            pl.BlockSpec((N // 5, 16), lambda i: (i, 0)),
            pl.BlockSpec((64, 32), lambda i: (0, 0)),
            pl.BlockSpec((1, 64), lambda i: (0, 0)),
        ],
        out_specs=pl.BlockSpec((N // 5, 32), lambda i: (i, 0)),
    )(acca, accb, hext, adext, w2f, b1)


def _stage_e_body(acca_ref, accb_ref, h2ext_ref, ad2ext_ref, b2_ref, out_ref):
    acc = acca_ref[...] + accb_ref[...]
    h2 = h2ext_ref[:, :7]
    wself = jnp.exp(_lrelu(h2ext_ref[:, 8:9] + ad2ext_ref[:, 0:1]))
    num = acc[:, :7] + wself * h2
    den = acc[:, 7:8] + wself
    o = num / (den + 1e-16) + b2_ref[...]
    o = o - jnp.max(o, axis=1, keepdims=True)
    out_ref[...] = o - jnp.log(jnp.sum(jnp.exp(o), axis=1, keepdims=True))


def _stage_e(acca, accb, h2ext, ad2ext, b2):
    return pl.pallas_call(
        _stage_e_body,
        out_shape=jax.ShapeDtypeStruct((N, 7), jnp.float32),
        grid=(5,),
        in_specs=[
            pl.BlockSpec((N // 5, 16), lambda i: (i, 0)),
            pl.BlockSpec((N // 5, 16), lambda i: (i, 0)),
            pl.BlockSpec((N // 5, 16), lambda i: (i, 0)),
            pl.BlockSpec((N // 5, 16), lambda i: (i, 0)),
            pl.BlockSpec((1, 7), lambda i: (0, 0)),
        ],
        out_specs=pl.BlockSpec((N // 5, 7), lambda i: (i, 0)),
    )(acca, accb, h2ext, ad2ext, b2)


@jax.jit
def kernel(x, edge_index, W1, a_src1, a_dst1, b1, W2, a_src2, a_dst2, b2):
    f32 = jnp.float32
    eye8 = jnp.eye(H1, dtype=f32)
    As = (eye8[:, None, :] * a_src1[:, :, None]).reshape(H1 * C1, H1)
    Ad = (eye8[:, None, :] * a_dst1[:, :, None]).reshape(H1 * C1, H1)
    wcat = jnp.concatenate(
        [W1, W1 @ As, W1 @ As, W1 @ Ad, W1 @ Ad], axis=1)
    v_s2 = W2 @ a_src2[0]
    v_d2 = W2 @ a_dst2[0]
    w2f = jnp.concatenate(
        [W2, jnp.zeros((H1 * C1, 1), f32),
         jnp.tile(v_s2[:, None], (1, 8)),
         jnp.tile(v_d2[:, None], (1, 16))], axis=1)

    e_in = edge_index.shape[1]
    src = edge_index[0].astype(jnp.int32)
    dst = edge_index[1].astype(jnp.int32)
    src_p = jnp.concatenate(
        [src, jnp.zeros((EPAD - e_in,), jnp.int32)]).reshape(NW, NCHUNK, CHUNK)
    dst_p = jnp.concatenate(
        [dst, jnp.full((EPAD - e_in,), N, jnp.int32)]).reshape(NW, NCHUNK, CHUNK)

    packed1 = _stage_a(x, wcat)
    hext1 = jnp.pad(packed1[:, :80], ((0, TPAD - N), (0, 0)))
    adext1 = jnp.pad(packed1[:, 80:96], ((0, TPAD - N), (0, 0)))
    acc1 = _edge_pass1(hext1, adext1, src_p, dst_p)
    packed2 = _stage_c(acc1[0, :N], acc1[1, :N], hext1[:N], adext1[:N], w2f,
                       b1.reshape(1, 64))
    h2ext = jnp.pad(packed2[:, :16], ((0, TPAD - N), (0, 0)))
    ad2ext = jnp.pad(packed2[:, 16:32], ((0, TPAD - N), (0, 0)))

    acc2 = _edge_pass2(h2ext, ad2ext, src_p, dst_p)
    return _stage_e(acc2[0, :N], acc2[1, :N], h2ext[:N], ad2ext[:N],
                    b2.reshape(1, 7))
